# baseline (speedup 1.0000x reference)
import jax
import jax.numpy as jnp
from jax.experimental import pallas as pl


def _body(f1_ref, f2_ref, w_ref, o_ref):
    f1 = f1_ref[...]
    f2 = f2_ref[...]
    bz, n1 = f1.shape
    n2 = f2.shape[1]
    big = (f1[:, :, None] * f2[:, None, :]).reshape(bz, n1 * n2)
    o_ref[...] = jnp.dot(big, w_ref[...], preferred_element_type=jnp.float32)


def kernel(features_1, features_2, mixing_matrix):
    z, n1 = features_1.shape
    n2 = features_2.shape[1]
    n_out = mixing_matrix.shape[0]
    wt = mixing_matrix.T
    bz = 256
    return pl.pallas_call(
        _body,
        grid=(z // bz,),
        in_specs=[
            pl.BlockSpec((bz, n1), lambda g: (g, 0)),
            pl.BlockSpec((bz, n2), lambda g: (g, 0)),
            pl.BlockSpec((n1 * n2, n_out), lambda g: (0, 0)),
        ],
        out_specs=pl.BlockSpec((bz, n_out), lambda g: (g, 0)),
        out_shape=jax.ShapeDtypeStruct((z, n_out), jnp.float32),
    )(features_1, features_2, wt)


# speedup vs baseline: 1.8455x; 1.8455x over previous
import jax
import jax.numpy as jnp
from jax.experimental import pallas as pl


def _body(f1_ref, f2_ref, w_ref, o_ref):
    f1 = f1_ref[...]
    f2 = f2_ref[...]
    bz, n1 = f1.shape
    n2 = f2.shape[1]
    big = (f1[:, :, None] * f2[:, None, :]).reshape(bz, n1 * n2)
    o_ref[...] = jnp.dot(big, w_ref[...], preferred_element_type=jnp.float32)


def kernel(features_1, features_2, mixing_matrix):
    z, n1 = features_1.shape
    n2 = features_2.shape[1]
    n_out = mixing_matrix.shape[0]
    wt = mixing_matrix.T.astype(jnp.bfloat16)
    f1 = features_1.astype(jnp.bfloat16)
    f2 = features_2.astype(jnp.bfloat16)
    bz = 512
    return pl.pallas_call(
        _body,
        grid=(z // bz,),
        in_specs=[
            pl.BlockSpec((bz, n1), lambda g: (g, 0)),
            pl.BlockSpec((bz, n2), lambda g: (g, 0)),
            pl.BlockSpec((n1 * n2, n_out), lambda g: (0, 0)),
        ],
        out_specs=pl.BlockSpec((bz, n_out), lambda g: (g, 0)),
        out_shape=jax.ShapeDtypeStruct((z, n_out), jnp.float32),
    )(f1, f2, wt)


# speedup vs baseline: 4.7373x; 2.5670x over previous
import jax
import jax.numpy as jnp
from jax.experimental import pallas as pl


def _body(f1_ref, f2_ref, w_ref, o_ref):
    f1t = f1_ref[...]
    f2t = f2_ref[...]
    n1, bz = f1t.shape
    n2 = f2t.shape[0]
    big = (f1t[:, None, :] * f2t[None, :, :]).reshape(n1 * n2, bz)
    o_ref[...] = jnp.dot(w_ref[...], big, preferred_element_type=jnp.float32)


def kernel(features_1, features_2, mixing_matrix):
    z, n1 = features_1.shape
    n2 = features_2.shape[1]
    n_out = mixing_matrix.shape[0]
    f1t = features_1.T.astype(jnp.bfloat16)
    f2t = features_2.T.astype(jnp.bfloat16)
    w = mixing_matrix.astype(jnp.bfloat16)
    bz = 512
    outt = pl.pallas_call(
        _body,
        grid=(z // bz,),
        in_specs=[
            pl.BlockSpec((n1, bz), lambda g: (0, g)),
            pl.BlockSpec((n2, bz), lambda g: (0, g)),
            pl.BlockSpec((n_out, n1 * n2), lambda g: (0, 0)),
        ],
        out_specs=pl.BlockSpec((n_out, bz), lambda g: (0, g)),
        out_shape=jax.ShapeDtypeStruct((n_out, z), jnp.float32),
    )(f1t, f2t, w)
    return outt.T


# speedup vs baseline: 5.0396x; 1.0638x over previous
import jax
import jax.numpy as jnp
from jax.experimental import pallas as pl


def _body(f1_ref, f2_ref, w_ref, o_ref):
    f1t = f1_ref[...].astype(jnp.bfloat16).T
    f2t = f2_ref[...].astype(jnp.bfloat16).T
    n1, bz = f1t.shape
    n2 = f2t.shape[0]
    w = w_ref[...].astype(jnp.bfloat16)
    big = (f1t[:, None, :] * f2t[None, :, :]).reshape(n1 * n2, bz)
    o_ref[...] = jnp.dot(w, big, preferred_element_type=jnp.float32)


def kernel(features_1, features_2, mixing_matrix):
    z, n1 = features_1.shape
    n2 = features_2.shape[1]
    n_out = mixing_matrix.shape[0]
    bz = 512
    outt = pl.pallas_call(
        _body,
        grid=(z // bz,),
        in_specs=[
            pl.BlockSpec((bz, n1), lambda g: (g, 0)),
            pl.BlockSpec((bz, n2), lambda g: (g, 0)),
            pl.BlockSpec((n_out, n1 * n2), lambda g: (0, 0)),
        ],
        out_specs=pl.BlockSpec((n_out, bz), lambda g: (0, g)),
        out_shape=jax.ShapeDtypeStruct((n_out, z), jnp.float32),
    )(features_1, features_2, mixing_matrix)
    return outt.T
